# Initial kernel scaffold; baseline (speedup 1.0000x reference)
import jax
import jax.numpy as jnp
from jax.experimental import pallas as pl


def kernel(hidden_states, router_w, w1, v1, w2):
    raise NotImplementedError("write your pallas kernel here")


# speedup vs baseline: 1.9156x; 1.9156x over previous
import jax
import jax.numpy as jnp
from jax.experimental import pallas as pl
from jax.experimental.pallas import tpu as pltpu


def _fused_moe_body(x_ref, rw_ref, w1_ref, v1_ref, w2_ref, out_ref,
                    idx2_ref, wv2_ref):
    e = pl.program_id(0)
    E = pl.num_programs(0)

    @pl.when(e == 0)
    def _route():
        x = x_ref[...]
        logits = jax.lax.dot_general(x, rw_ref[...], (((1,), (1,)), ((), ())))
        m = jnp.max(logits, axis=-1, keepdims=True)
        s = jnp.exp(logits - m)
        p = s / jnp.sum(s, axis=-1, keepdims=True)
        lane = jax.lax.broadcasted_iota(jnp.int32, p.shape, 1)
        m1 = jnp.max(p, axis=-1, keepdims=True)
        i1 = jnp.min(jnp.where(p == m1, lane, E), axis=-1, keepdims=True)
        p2 = jnp.where(lane == i1, -jnp.inf, p)
        m2 = jnp.max(p2, axis=-1, keepdims=True)
        i2 = jnp.min(jnp.where(p2 == m2, lane, E), axis=-1, keepdims=True)
        denom = m1 + m2
        idx2_ref[:, 0:1] = i1
        idx2_ref[:, 1:2] = i2
        wv2_ref[:, 0:1] = m1 / denom
        wv2_ref[:, 1:2] = m2 / denom
        out_ref[...] = jnp.zeros_like(out_ref)

    x = x_ref[...]
    w_e = (jnp.where(idx2_ref[:, 0:1] == e, wv2_ref[:, 0:1], 0.0)
           + jnp.where(idx2_ref[:, 1:2] == e, wv2_ref[:, 1:2], 0.0))
    h1 = jax.lax.dot_general(x, w1_ref[0], (((1,), (1,)), ((), ())))
    h2 = jax.lax.dot_general(x, v1_ref[0], (((1,), (1,)), ((), ())))
    g = (h1 * jax.nn.sigmoid(h1)) * h2
    y = jax.lax.dot_general(g, w2_ref[0], (((1,), (0,)), ((), ())))
    out_ref[...] += w_e * y


def kernel(hidden_states, router_w, w1, v1, w2):
    B, S, H = hidden_states.shape
    E, F, _ = w1.shape
    x = jnp.transpose(hidden_states, (1, 0, 2)).reshape(-1, H)
    T = x.shape[0]

    out = pl.pallas_call(
        _fused_moe_body,
        grid=(E,),
        in_specs=[
            pl.BlockSpec((T, H), lambda e: (0, 0)),
            pl.BlockSpec((E, H), lambda e: (0, 0)),
            pl.BlockSpec((1, F, H), lambda e: (e, 0, 0)),
            pl.BlockSpec((1, F, H), lambda e: (e, 0, 0)),
            pl.BlockSpec((1, F, H), lambda e: (e, 0, 0)),
        ],
        out_specs=pl.BlockSpec((T, H), lambda e: (0, 0)),
        out_shape=jax.ShapeDtypeStruct((T, H), jnp.float32),
        scratch_shapes=[
            pltpu.VMEM((T, 2), jnp.int32),
            pltpu.VMEM((T, 2), jnp.float32),
        ],
        compiler_params=pltpu.CompilerParams(
            dimension_semantics=("arbitrary",),
        ),
    )(x, router_w, w1, v1, w2)

    return jnp.transpose(out.reshape(S, B, H), (1, 0, 2))
